# baseline (speedup 1.0000x reference)
import functools

import jax
import jax.numpy as jnp
from jax import lax
from jax.experimental import pallas as pl
from jax.experimental.pallas import tpu as pltpu
from jax.experimental.pallas import tpu_sc as plsc

_NC = 2
_NS = 16
_NW = _NC * _NS

_EPC = 4
_NB = 8


def _comb_body(lt_ref, pt_ref, w_ref, b_ref, comb_ref):
    e = pt_ref[...][:, None, :] + lt_ref[...][None, :, :]
    mu = jnp.mean(e, axis=-1, keepdims=True)
    var = jnp.mean((e - mu) ** 2, axis=-1, keepdims=True)
    normed = (e - mu) / jnp.sqrt(var + 1e-5)
    comb_ref[...] = normed * w_ref[...][None, :, :] + b_ref[...][None, :, :]


def _idx_body(x_ref, idx_ref):
    s = lax.broadcasted_iota(jnp.int32, x_ref.shape, 1)
    idx_ref[...] = x_ref[...] + s * 98


def _make_gather(batch, seq, d, n_chunks, elems_per_w):
    ch = _EPC * seq
    mesh = plsc.VectorSubcoreMesh(core_axis_name="c", subcore_axis_name="s")

    @functools.partial(
        pl.kernel,
        mesh=mesh,
        compiler_params=pltpu.CompilerParams(use_tc_tiling_on_sc=True),
        out_type=jax.ShapeDtypeStruct((batch, seq, d), jnp.float32),
        scratch_types=[
            pltpu.VMEM((n_chunks, ch), jnp.int32),
            *[pltpu.VMEM((ch, d), jnp.float32) for _ in range(_NB)],
            *[pltpu.SemaphoreType.DMA for _ in range(2 * _NB)],
        ],
    )
    def gather_kernel(comb_hbm, idx_hbm, out_hbm, idx_v, *rest):
        bufs = rest[:_NB]
        gsems = rest[_NB:2 * _NB]
        ssems = rest[2 * _NB:]
        wid = lax.axis_index("s") * _NC + lax.axis_index("c")
        ebase = wid * elems_per_w
        pltpu.sync_copy(idx_hbm.at[wid], idx_v)

        def scatter_descs(b, c):
            return [
                pltpu.make_async_copy(
                    bufs[b].at[pl.ds(e * seq, seq)],
                    out_hbm.at[ebase + c * _EPC + e],
                    ssems[b],
                )
                for e in range(_EPC)
            ]

        def body(j, carry):
            gds = []
            for b in range(_NB):
                c = j * _NB + b

                @pl.when(j > 0)
                def _drain(b=b, c=c):
                    for dsc in scatter_descs(b, c):
                        dsc.wait()

                dcp = pltpu.make_async_copy(
                    comb_hbm.at[idx_v.at[c]], bufs[b], gsems[b]
                )
                dcp.start()
                gds.append(dcp)
            for b in range(_NB):
                c = j * _NB + b
                gds[b].wait()
                for dsc in scatter_descs(b, c):
                    dsc.start()
            return carry

        lax.fori_loop(0, n_chunks // _NB, body, 0)
        for b in range(_NB):
            for dsc in scatter_descs(b, 0):
                dsc.wait()

    return gather_kernel


_NPART = 4


def kernel(x, letter_table, pos_table, ln_w, ln_b):
    batch, seq = x.shape
    nchar, d = letter_table.shape
    pbatch = batch // _NPART
    elems_per_w = pbatch // _NW
    n_chunks = elems_per_w // _EPC

    comb = pl.pallas_call(
        _comb_body,
        out_shape=jax.ShapeDtypeStruct((seq, nchar, d), jnp.float32),
    )(
        letter_table,
        pos_table[:seq],
        ln_w.reshape(1, d),
        ln_b.reshape(1, d),
    )

    xb = 1024
    idx2d = pl.pallas_call(
        _idx_body,
        grid=(batch // xb,),
        in_specs=[pl.BlockSpec((xb, seq), lambda i: (i, 0))],
        out_specs=pl.BlockSpec((xb, seq), lambda i: (i, 0)),
        out_shape=jax.ShapeDtypeStruct((batch, seq), jnp.int32),
    )(x)

    comb2 = comb.reshape(seq * nchar, d)
    idx4 = idx2d.reshape(_NPART, _NW, n_chunks, _EPC * seq)
    gather_fn = _make_gather(pbatch, seq, d, n_chunks, elems_per_w)
    parts = [gather_fn(comb2, idx4[p]) for p in range(_NPART)]
    return jnp.concatenate(parts, axis=0)


# speedup vs baseline: 1.6233x; 1.6233x over previous
import functools

import jax
import jax.numpy as jnp
from jax import lax
from jax.experimental import pallas as pl
from jax.experimental.pallas import tpu as pltpu
from jax.experimental.pallas import tpu_sc as plsc

_NC = 2
_NS = 16
_NW = _NC * _NS

_EPC = 4
_NB = 8


def _comb_body(lt_ref, pt_ref, w_ref, b_ref, comb_ref):
    e = pt_ref[...][:, None, :] + lt_ref[...][None, :, :]
    mu = jnp.mean(e, axis=-1, keepdims=True)
    var = jnp.mean((e - mu) ** 2, axis=-1, keepdims=True)
    normed = (e - mu) / jnp.sqrt(var + 1e-5)
    comb_ref[...] = normed * w_ref[...][None, :, :] + b_ref[...][None, :, :]


def _idx_body(x_ref, idx_ref):
    s = lax.broadcasted_iota(jnp.int32, x_ref.shape, 1)
    idx_ref[...] = x_ref[...] + s * 98


def _make_gather(batch, seq, d, n_chunks, elems_per_w):
    ch = _EPC * seq
    mesh = plsc.VectorSubcoreMesh(core_axis_name="c", subcore_axis_name="s")

    @functools.partial(
        pl.kernel,
        mesh=mesh,
        compiler_params=pltpu.CompilerParams(use_tc_tiling_on_sc=True),
        out_type=jax.ShapeDtypeStruct((batch, seq, d), jnp.float32),
        scratch_types=[
            pltpu.VMEM((n_chunks, ch), jnp.int32),
            *[pltpu.VMEM((ch, d), jnp.float32) for _ in range(_NB)],
            *[pltpu.SemaphoreType.DMA for _ in range(2 * _NB)],
        ],
    )
    def gather_kernel(comb_hbm, idx_hbm, out_hbm, idx_v, *rest):
        bufs = rest[:_NB]
        gsems = rest[_NB:2 * _NB]
        ssems = rest[2 * _NB:]
        wid = lax.axis_index("s") * _NC + lax.axis_index("c")
        ebase = wid * elems_per_w
        pltpu.sync_copy(idx_hbm.at[wid], idx_v)

        def scatter_descs(b, c):
            return [
                pltpu.make_async_copy(
                    bufs[b].at[pl.ds(e * seq, seq)],
                    out_hbm.at[ebase + c * _EPC + e],
                    ssems[b],
                )
                for e in range(_EPC)
            ]

        def body(j, carry):
            gds = []
            for b in range(_NB):
                c = j * _NB + b

                @pl.when(j > 0)
                def _drain(b=b, c=c):
                    for dsc in scatter_descs(b, c):
                        dsc.wait()

                dcp = pltpu.make_async_copy(
                    comb_hbm.at[idx_v.at[c]], bufs[b], gsems[b]
                )
                dcp.start()
                gds.append(dcp)
            for b in range(_NB):
                c = j * _NB + b
                gds[b].wait()
                for dsc in scatter_descs(b, c):
                    dsc.start()
            return carry

        lax.fori_loop(0, n_chunks // _NB, body, 0)
        for b in range(_NB):
            for dsc in scatter_descs(b, 0):
                dsc.wait()

    return gather_kernel


_NPART = 1


def kernel(x, letter_table, pos_table, ln_w, ln_b):
    batch, seq = x.shape
    nchar, d = letter_table.shape
    pbatch = batch // _NPART
    elems_per_w = pbatch // _NW
    n_chunks = elems_per_w // _EPC

    comb = pl.pallas_call(
        _comb_body,
        out_shape=jax.ShapeDtypeStruct((seq, nchar, d), jnp.float32),
    )(
        letter_table,
        pos_table[:seq],
        ln_w.reshape(1, d),
        ln_b.reshape(1, d),
    )

    xb = 1024
    idx2d = pl.pallas_call(
        _idx_body,
        grid=(batch // xb,),
        in_specs=[pl.BlockSpec((xb, seq), lambda i: (i, 0))],
        out_specs=pl.BlockSpec((xb, seq), lambda i: (i, 0)),
        out_shape=jax.ShapeDtypeStruct((batch, seq), jnp.int32),
    )(x)

    comb2 = comb.reshape(seq * nchar, d)
    idx4 = idx2d.reshape(_NPART, _NW, n_chunks, _EPC * seq)
    gather_fn = _make_gather(pbatch, seq, d, n_chunks, elems_per_w)
    parts = [gather_fn(comb2, idx4[p]) for p in range(_NPART)]
    return parts[0] if _NPART == 1 else jnp.concatenate(parts, axis=0)
